# Initial kernel scaffold; baseline (speedup 1.0000x reference)
import jax
import jax.numpy as jnp
from jax.experimental import pallas as pl


def kernel(adv_pc, ori_pc):
    raise NotImplementedError("write your pallas kernel here")


# speedup vs baseline: 24.9597x; 24.9597x over previous
import functools

import jax
import jax.numpy as jnp
from jax.experimental import pallas as pl
from jax.experimental.pallas import tpu as pltpu

KNN_K = 5
KNN_ALPHA = 1.05
W_CHAMFER = 5.0
W_KNN = 3.0


def _loss_kernel(adv_ref, advt_ref, orit_ref, cham_ref, knn_ref, vscr_ref,
                 *, blk, n, nb):
    rb = pl.program_id(1)

    a_blk = adv_ref[0]
    a_t = advt_ref[0]
    o_t = orit_ref[0]

    aa = jnp.sum(a_blk * a_blk, axis=1, keepdims=True)
    oo = jnp.sum(o_t * o_t, axis=0, keepdims=True)
    sa = jnp.sum(a_t * a_t, axis=0, keepdims=True)

    a_blk16 = a_blk.astype(jnp.bfloat16)
    a_t16 = a_t.astype(jnp.bfloat16)
    o_t16 = o_t.astype(jnp.bfloat16)

    zo = jnp.dot(a_blk16, o_t16, preferred_element_type=jnp.float32)
    qo = aa + oo - 2.0 * zo
    mo = jnp.min(qo, axis=1, keepdims=True)
    csum = jnp.sum(mo)

    za = jnp.dot(a_blk16, a_t16, preferred_element_type=jnp.float32)
    d = jnp.maximum(aa + sa - 2.0 * za, -1e-6)

    t = jnp.full((blk, 1), -jnp.inf, jnp.float32)
    c = jnp.zeros((blk, 1), jnp.float32)
    s = jnp.zeros((blk, 1), jnp.float32)
    m0 = None
    for step in range(KNN_K + 1):
        cand = jnp.where(d > t, d, jnp.inf)
        m = jnp.min(cand, axis=1, keepdims=True)
        if step == 0:
            m0 = m
        cnt = jnp.sum(jnp.where(d == m, 1.0, 0.0), axis=1, keepdims=True)
        live = c < (KNN_K + 1)
        c = jnp.where(live, c + cnt, c)
        s = jnp.where(live, s + m * cnt, s)
        t = jnp.where(live, m, t)
    sum6 = s - (c - (KNN_K + 1)) * t
    value = (sum6 - m0) / KNN_K

    vscr_ref[pl.ds(rb * blk, blk), :] = value

    prev = cham_ref[:, :, :]
    acc = jnp.where(rb == 0, csum, prev + csum)
    cham_ref[:, :, :] = jnp.where(rb == nb - 1, acc / n, acc)

    @pl.when(rb == nb - 1)
    def _knn_stats():
        v = vscr_ref[:, :]
        mean = jnp.sum(v) / n
        diff = v - mean
        var = jnp.sum(diff * diff) / (n - 1)
        thr = mean + KNN_ALPHA * jnp.sqrt(var)
        masked = jnp.where(v > thr, v, 0.0)
        knn_ref[:, :, :] = (jnp.sum(masked) / n).reshape(1, 1, 1)


def kernel(adv_pc, ori_pc):
    b, n, _ = adv_pc.shape
    blk = 256
    nb = n // blk
    adv_t = adv_pc.transpose(0, 2, 1)
    ori_t = ori_pc.transpose(0, 2, 1)
    cham, knn = pl.pallas_call(
        functools.partial(_loss_kernel, blk=blk, n=n, nb=nb),
        grid=(b, nb),
        in_specs=[
            pl.BlockSpec((1, blk, 3), lambda i, r: (i, r, 0)),
            pl.BlockSpec((1, 3, n), lambda i, r: (i, 0, 0)),
            pl.BlockSpec((1, 3, n), lambda i, r: (i, 0, 0)),
        ],
        out_specs=[
            pl.BlockSpec((1, 1, 1), lambda i, r: (i, 0, 0)),
            pl.BlockSpec((1, 1, 1), lambda i, r: (i, 0, 0)),
        ],
        out_shape=[
            jax.ShapeDtypeStruct((b, 1, 1), jnp.float32),
            jax.ShapeDtypeStruct((b, 1, 1), jnp.float32),
        ],
        scratch_shapes=[pltpu.VMEM((n, 1), jnp.float32)],
        compiler_params=pltpu.CompilerParams(
            dimension_semantics=("parallel", "arbitrary")),
    )(adv_pc, adv_t, ori_t)
    return jnp.mean(cham) * W_CHAMFER + jnp.mean(knn) * W_KNN
